# baseline (speedup 1.0000x reference)
import functools

import jax
import jax.numpy as jnp
import numpy as np
from jax import lax
from jax.experimental import pallas as pl
from jax.experimental.pallas import tpu as pltpu
from jax.experimental.pallas import tpu_sc as plsc

_N = 10000
_E = 320000
_D = 128
_DE = 16
_H = 8
_DH = _D // _H
_K = int(np.ceil(0.8 * _E))

_NC = 2
_NS = 16
_NW = _NC * _NS


@functools.lru_cache(maxsize=None)
def _sc_gather_rows(V, Dw, B, dtype_name, W):
    dtype = jnp.dtype(dtype_name)
    b_per_w = B // _NW
    nwin = b_per_w // W
    npairs = nwin // 2
    leftover = nwin % 2
    assert b_per_w % W == 0 and W % 8 == 0 and W <= 128
    mesh = plsc.VectorSubcoreMesh(core_axis_name="c", subcore_axis_name="s")

    @functools.partial(
        pl.kernel,
        out_type=jax.ShapeDtypeStruct((B, Dw), dtype),
        mesh=mesh,
        compiler_params=pltpu.CompilerParams(use_tc_tiling_on_sc=(Dw % 128 == 0)),
        scratch_types=[
            pltpu.VMEM((W,), jnp.int32),
            pltpu.VMEM((W,), jnp.int32),
            pltpu.VMEM((2, W, Dw), dtype),
            pltpu.SemaphoreType.DMA,
            pltpu.SemaphoreType.DMA,
            pltpu.SemaphoreType.DMA,
            pltpu.SemaphoreType.DMA,
        ],
    )
    def k(table_hbm, idx_hbm, out_hbm, idx_v0, idx_v1, rows_v,
          semi, semg, semo0, semo1):
        wid = lax.axis_index("s") * _NC + lax.axis_index("c")
        idxv = (idx_v0, idx_v1)
        semo = (semo0, semo1)

        def start_i(w, b):
            base = wid * b_per_w + w * W
            pltpu.async_copy(idx_hbm.at[pl.ds(base, W)], idxv[b], semi)

        def wait_i(b):
            pltpu.make_async_copy(idx_hbm.at[pl.ds(0, W)], idxv[b],
                                  semi).wait()

        def gather_store(w, b):
            base = wid * b_per_w + w * W
            pltpu.async_copy(table_hbm.at[idxv[b]], rows_v.at[b], semg).wait()
            pltpu.async_copy(rows_v.at[b], out_hbm.at[pl.ds(base, W)], semo[b])

        def wait_store(b):
            pltpu.make_async_copy(rows_v.at[b], out_hbm.at[pl.ds(0, W)],
                                  semo[b]).wait()

        start_i(0, 0)

        def pair(g, carry):
            w0 = 2 * g
            wait_i(0)

            @pl.when(w0 + 1 < nwin)
            def _():
                start_i(w0 + 1, 1)

            @pl.when(g > 0)
            def _():
                wait_store(0)

            gather_store(w0, 0)

            @pl.when(w0 + 1 < nwin)
            def _():
                wait_i(1)

                @pl.when(w0 + 2 < nwin)
                def _():
                    start_i(w0 + 2, 0)

                @pl.when(g > 0)
                def _():
                    wait_store(1)

                gather_store(w0 + 1, 1)

            return carry

        lax.fori_loop(0, (nwin + 1) // 2, pair, 0)
        wait_store(0)
        if nwin > 1:
            wait_store(1)

    return k


@functools.lru_cache(maxsize=None)
def _sc_scatter_add_rows(V, Dw, B, W):
    b_per_w = B // _NW
    nwin = b_per_w // W
    assert b_per_w % W == 0 and W % 8 == 0 and W <= 128
    mesh = plsc.VectorSubcoreMesh(core_axis_name="c", subcore_axis_name="s")

    @functools.partial(
        pl.kernel,
        out_type=jax.ShapeDtypeStruct((_NC, V, Dw), jnp.float32),
        mesh=mesh,
        compiler_params=pltpu.CompilerParams(use_tc_tiling_on_sc=(Dw % 128 == 0)),
        scratch_types=[
            pltpu.VMEM((W,), jnp.int32),
            pltpu.VMEM((W,), jnp.int32),
            pltpu.VMEM((2, W, Dw), jnp.float32),
            pltpu.VMEM_SHARED((V, Dw), jnp.float32),
            pltpu.SemaphoreType.DMA,
            pltpu.SemaphoreType.DMA,
        ],
    )
    def k(upd_hbm, idx_hbm, zero_hbm, out_hbm, idx_v0, idx_v1, upd_v, acc_sh,
          semi, semu):
        cid = lax.axis_index("c")
        sid = lax.axis_index("s")
        wid = sid * _NC + cid
        idxv = (idx_v0, idx_v1)

        @pl.when(sid == 0)
        def _():
            pltpu.sync_copy(zero_hbm, acc_sh)

        plsc.subcore_barrier()

        def start_iu(w, b):
            base = wid * b_per_w + w * W
            pltpu.async_copy(idx_hbm.at[pl.ds(base, W)], idxv[b], semi)
            pltpu.async_copy(upd_hbm.at[pl.ds(base, W)], upd_v.at[b], semu)

        def wait_iu(b):
            pltpu.make_async_copy(idx_hbm.at[pl.ds(0, W)], idxv[b],
                                  semi).wait()
            pltpu.make_async_copy(upd_hbm.at[pl.ds(0, W)], upd_v.at[b],
                                  semu).wait()

        def scat(b):
            pltpu.sync_copy(upd_v.at[b], acc_sh.at[idxv[b]], add=True)

        start_iu(0, 0)

        def pair(g, carry):
            w0 = 2 * g
            wait_iu(0)

            @pl.when(w0 + 1 < nwin)
            def _():
                start_iu(w0 + 1, 1)

            scat(0)

            @pl.when(w0 + 1 < nwin)
            def _():
                wait_iu(1)

                @pl.when(w0 + 2 < nwin)
                def _():
                    start_iu(w0 + 2, 0)

                scat(1)

            return carry

        lax.fori_loop(0, (nwin + 1) // 2, pair, 0)
        plsc.subcore_barrier()

        @pl.when(sid == 0)
        def _():
            pltpu.sync_copy(acc_sh, out_hbm.at[cid])

    return k


_SR = 4096
_SC_ = 128
_S = _SR * _SC_
_NBITS = 19


def _sort_schedule():
    ds, sb = [], []
    for s in range(1, _NBITS + 1):
        d = 1 << (s - 1)
        while d >= 1:
            ds.append(d)
            sb.append(1 << s)
            d //= 2
    return np.array(ds, np.int32), np.array(sb, np.int32)


def _sort_body(score_ref, dsched_ref, out_ref, key_ref):
    rows = lax.broadcasted_iota(jnp.int32, (_SR, _SC_), 0)
    cols = lax.broadcasted_iota(jnp.int32, (_SR, _SC_), 1)
    ig = rows + _SR * cols
    b = pltpu.bitcast(score_ref[...], jnp.int32)
    key = jnp.where(b >= 0, jnp.int32(0x7FFFFFFF) - b, b) ^ jnp.int32(-2147483648)
    key_ref[...] = key
    out_ref[...] = ig

    nsteps = dsched_ref.shape[0] // 2

    def step(t, carry):
        d = dsched_ref[2 * t]
        sblk = dsched_ref[2 * t + 1]
        ai = key_ref[...]
        ix = out_ref[...]
        first = (ig & d) == 0
        asc = (ig & sblk) == 0
        keep_small = first == asc

        def row_case(ai, ix):
            return (
                pltpu.roll(ai, _SR - d, 0), pltpu.roll(ai, d, 0),
                pltpu.roll(ix, _SR - d, 0), pltpu.roll(ix, d, 0),
            )

        def col_case(ai, ix):
            m = d >> 12
            return (
                pltpu.roll(ai, _SC_ - m, 1), pltpu.roll(ai, m, 1),
                pltpu.roll(ix, _SC_ - m, 1), pltpu.roll(ix, m, 1),
            )

        fa, ba, fi, bi = lax.cond(d < _SR, row_case, col_case, ai, ix)
        pa = jnp.where(first, fa, ba)
        pi = jnp.where(first, fi, bi)
        mine_less = (ai < pa) | ((ai == pa) & (ix < pi))
        take = keep_small ^ mine_less
        key_ref[...] = jnp.where(take, pa, ai)
        out_ref[...] = jnp.where(take, pi, ix)
        return carry

    lax.fori_loop(0, nsteps, step, 0)


def _bitonic_argsort(score):
    pad = jnp.full((_S - _E,), -jnp.inf, jnp.float32)
    s2 = jnp.concatenate([score, pad]).reshape(_SC_, _SR).T
    ds, sb = _sort_schedule()
    sched = jnp.asarray(np.stack([ds, sb], 1).reshape(-1))
    idx2d, _ = pl.pallas_call(
        _sort_body,
        out_shape=(
            jax.ShapeDtypeStruct((_SR, _SC_), jnp.int32),
            jax.ShapeDtypeStruct((_SR, _SC_), jnp.int32),
        ),
        in_specs=[
            pl.BlockSpec(memory_space=pltpu.VMEM),
            pl.BlockSpec(memory_space=pltpu.SMEM),
        ],
        out_specs=(
            pl.BlockSpec(memory_space=pltpu.VMEM),
            pl.BlockSpec(memory_space=pltpu.VMEM),
        ),
    )(s2, sched)
    return idx2d.T.reshape(-1)


_BE2 = 8000


def _edge2_body(hsrc_ref, p_ref, g0_ref, g1_ref, rep_ref, out_ref):
    denom = g0_ref[...] + g1_ref[...] + jnp.float32(1e-16)
    alpha16 = p_ref[...] / denom
    afull = jnp.dot(alpha16, rep_ref[...], preferred_element_type=jnp.float32)
    out_ref[...] = hsrc_ref[...] * afull


def _edge2(hsrc, p16, gs0, gs1, rep):
    grid = _E // _BE2
    return pl.pallas_call(
        _edge2_body,
        grid=(grid,),
        in_specs=[
            pl.BlockSpec((_BE2, _D), lambda i: (i, 0)),
            pl.BlockSpec((_BE2, 16), lambda i: (i, 0)),
            pl.BlockSpec((_BE2, 16), lambda i: (i, 0)),
            pl.BlockSpec((_BE2, 16), lambda i: (i, 0)),
            pl.BlockSpec((16, _D), lambda i: (0, 0)),
        ],
        out_specs=pl.BlockSpec((_BE2, _D), lambda i: (i, 0)),
        out_shape=jax.ShapeDtypeStruct((_E, _D), jnp.float32),
    )(hsrc, p16, gs0, gs1, rep)


_BNF = 2000


def _final_body(ai_ref, bi_ref, aj_ref, bj_ref, wg_ref, bg_ref, w1_ref, w2_ref,
                out_ref):
    xi = ai_ref[0] + ai_ref[1] + bi_ref[...]
    xj = aj_ref[0] + aj_ref[1] + bj_ref[...]
    cat = jnp.concatenate([xi, xj], axis=1)
    g = jax.nn.sigmoid(
        jnp.dot(cat, wg_ref[...], preferred_element_type=jnp.float32)
        + bg_ref[...])
    fusion = (g * jnp.dot(xi, w1_ref[...], preferred_element_type=jnp.float32)
              + (1.0 - g) * jnp.dot(xj, w2_ref[...],
                                    preferred_element_type=jnp.float32))
    out_ref[0] = fusion + xi
    out_ref[1] = fusion + xj


def _final(acc_i, bout_i, acc_j, bout_j, Wg, bg, W1, W2):
    grid = _N // _BNF
    return pl.pallas_call(
        _final_body,
        grid=(grid,),
        in_specs=[
            pl.BlockSpec((2, _BNF, _D), lambda i: (0, i, 0)),
            pl.BlockSpec((1, _D), lambda i: (0, 0)),
            pl.BlockSpec((2, _BNF, _D), lambda i: (0, i, 0)),
            pl.BlockSpec((1, _D), lambda i: (0, 0)),
            pl.BlockSpec((2 * _D, _D), lambda i: (0, 0)),
            pl.BlockSpec((1, _D), lambda i: (0, 0)),
            pl.BlockSpec((_D, _D), lambda i: (0, 0)),
            pl.BlockSpec((_D, _D), lambda i: (0, 0)),
        ],
        out_specs=pl.BlockSpec((2, _BNF, _D), lambda i: (0, i, 0)),
        out_shape=jax.ShapeDtypeStruct((2, _N, _D), jnp.float32),
    )(acc_i, bout_i.reshape(1, _D), acc_j, bout_j.reshape(1, _D),
      Wg, bg.reshape(1, _D), W1, W2)


def _tree_sum(t):
    while t.shape[-1] > 1:
        t = t[..., 0::2] + t[..., 1::2]
    return t[..., 0]


def _fold_sum(t):
    while t.shape[-1] > 1:
        m = t.shape[-1] // 2
        t = t[..., :m] + t[..., m:]
    return t[..., 0]


def _node_embed(x, gamma, beta, Wx, bx):
    mu = jnp.mean(x, axis=0)
    var = jnp.var(x, axis=0)
    xn = (x - mu) / jnp.sqrt(var + 1e-5) * gamma + beta
    return xn @ Wx + bx


def _score_path(hsrc, hdst, asrc, adst, ea, We, ae):
    t1 = _tree_sum(hsrc.reshape(_E, _H, _DH) * asrc)
    t2 = _tree_sum(hdst.reshape(_E, _H, _DH) * adst)
    he = (ea @ We).reshape(_E, _H, _DH)
    t3 = _fold_sum(he * ae)
    logits = jax.nn.leaky_relu((t1 + t2) + t3, 0.2)
    score = _fold_sum(logits) / jnp.float32(8.0)
    p8 = jnp.exp(logits)
    p16 = jnp.concatenate([p8, p8], axis=1)
    return p16, score


_W = 80


def kernel(x_intra, edge_index_intra, edge_attr_intra, batch_ei_intra,
           x_inter, edge_index_inter, edge_attr_inter, batch_ei_inter,
           gamma_i, beta_i, Wx_i, bx_i, We_i, asrc_i, adst_i, ae_i, bout_i,
           gamma_j, beta_j, Wx_j, bx_j, We_j, asrc_j, adst_j, ae_j, bout_j,
           Wg, bg, W1, W2):
    rep16 = np.zeros((16, _D), np.float32)
    for hh in range(_H):
        rep16[hh, hh * _DH:(hh + 1) * _DH] = 1.0
    rep16 = jnp.asarray(rep16)

    src_i, dst_i = edge_index_intra[0], edge_index_intra[1]
    src_j, dst_j = edge_index_inter[0], edge_index_inter[1]

    h_i = _node_embed(x_intra, gamma_i, beta_i, Wx_i, bx_i)
    h_j = _node_embed(x_inter, gamma_j, beta_j, Wx_j, bx_j)

    hcat = jnp.concatenate([h_i, h_j], axis=0)
    gidx = jnp.concatenate([src_i, dst_i, src_j + _N, dst_j + _N])
    big = _sc_gather_rows(2 * _N, _D, 4 * _E, "float32", _W)(hcat, gidx)
    hsrc_i, hdst_i = big[:_E], big[_E:2 * _E]
    hsrc_j, hdst_j = big[2 * _E:3 * _E], big[3 * _E:]

    p16_i, score_i = _score_path(hsrc_i, hdst_i, asrc_i, adst_i,
                                 edge_attr_intra, We_i, ae_i)
    p16_j, score_j = _score_path(hsrc_j, hdst_j, asrc_j, adst_j,
                                 edge_attr_inter, We_j, ae_j)

    upd2 = jnp.concatenate([p16_i, p16_j], axis=0)
    sidx = jnp.concatenate([dst_i, dst_j + _N])
    zeros216 = jnp.zeros((2 * _N, 16), jnp.float32)
    ssum = _sc_scatter_add_rows(2 * _N, 16, 2 * _E, _W)(upd2, sidx, zeros216)
    stab = jnp.concatenate([ssum[0], ssum[1]], axis=0)
    g4 = jnp.concatenate(
        [dst_i, dst_i + 2 * _N, dst_j + _N, dst_j + 3 * _N])
    gs = _sc_gather_rows(4 * _N, 16, 4 * _E, "float32", _W)(stab, g4)
    gs0_i, gs1_i = gs[:_E], gs[_E:2 * _E]
    gs0_j, gs1_j = gs[2 * _E:3 * _E], gs[3 * _E:]

    zeros128 = jnp.zeros((_N, _D), jnp.float32)
    upd_i = _edge2(hsrc_i, p16_i, gs0_i, gs1_i, rep16)
    acc_i = _sc_scatter_add_rows(_N, _D, _E, _W)(
        upd_i, dst_i, zeros128)
    upd_j = _edge2(hsrc_j, p16_j, gs0_j, gs1_j, rep16)
    acc_j = _sc_scatter_add_rows(_N, _D, _E, _W)(
        upd_j, dst_j, zeros128)

    out = _final(acc_i, bout_i, acc_j, bout_j, Wg, bg, W1, W2)

    idx_i = _bitonic_argsort(score_i)[:_K]
    idx_j = _bitonic_argsort(score_j)[:_K]
    eitab = jnp.concatenate([
        jnp.concatenate([edge_index_intra.T.astype(jnp.int32),
                         jnp.zeros((_E, 14), jnp.int32)], axis=1),
        jnp.concatenate([edge_index_inter.T.astype(jnp.int32),
                         jnp.zeros((_E, 14), jnp.int32)], axis=1)], axis=0)
    eidx = jnp.concatenate([idx_i, idx_j + _E])
    eig = _sc_gather_rows(2 * _E, 16, 2 * _K, "int32", _W)(eitab, eidx)
    ei_i = eig[:_K, :2].T
    ei_j = eig[_K:, :2].T

    return (out, ei_i, ei_j)


# speedup vs baseline: 1.0546x; 1.0546x over previous
import functools

import jax
import jax.numpy as jnp
import numpy as np
from jax import lax
from jax.experimental import pallas as pl
from jax.experimental.pallas import tpu as pltpu
from jax.experimental.pallas import tpu_sc as plsc

_N = 10000
_E = 320000
_D = 128
_DE = 16
_H = 8
_DH = _D // _H
_K = int(np.ceil(0.8 * _E))

_NC = 2
_NS = 16
_NW = _NC * _NS


@functools.lru_cache(maxsize=None)
def _sc_gather_rows(V, Dw, B, dtype_name, W):
    dtype = jnp.dtype(dtype_name)
    b_per_w = B // _NW
    nwin = b_per_w // W
    npairs = nwin // 2
    leftover = nwin % 2
    assert b_per_w % W == 0 and W % 8 == 0 and W <= 128
    mesh = plsc.VectorSubcoreMesh(core_axis_name="c", subcore_axis_name="s")

    @functools.partial(
        pl.kernel,
        out_type=jax.ShapeDtypeStruct((B, Dw), dtype),
        mesh=mesh,
        compiler_params=pltpu.CompilerParams(use_tc_tiling_on_sc=(Dw % 128 == 0)),
        scratch_types=[
            pltpu.VMEM((W,), jnp.int32),
            pltpu.VMEM((W,), jnp.int32),
            pltpu.VMEM((2, W, Dw), dtype),
            pltpu.SemaphoreType.DMA,
            pltpu.SemaphoreType.DMA,
            pltpu.SemaphoreType.DMA,
            pltpu.SemaphoreType.DMA,
        ],
    )
    def k(table_hbm, idx_hbm, out_hbm, idx_v0, idx_v1, rows_v,
          semi, semg, semo0, semo1):
        wid = lax.axis_index("s") * _NC + lax.axis_index("c")
        idxv = (idx_v0, idx_v1)
        semo = (semo0, semo1)

        def start_i(w, b):
            base = wid * b_per_w + w * W
            pltpu.async_copy(idx_hbm.at[pl.ds(base, W)], idxv[b], semi)

        def wait_i(b):
            pltpu.make_async_copy(idx_hbm.at[pl.ds(0, W)], idxv[b],
                                  semi).wait()

        def gather_store(w, b):
            base = wid * b_per_w + w * W
            pltpu.async_copy(table_hbm.at[idxv[b]], rows_v.at[b], semg).wait()
            pltpu.async_copy(rows_v.at[b], out_hbm.at[pl.ds(base, W)], semo[b])

        def wait_store(b):
            pltpu.make_async_copy(rows_v.at[b], out_hbm.at[pl.ds(0, W)],
                                  semo[b]).wait()

        start_i(0, 0)

        def pair(g, carry):
            w0 = 2 * g
            wait_i(0)

            @pl.when(w0 + 1 < nwin)
            def _():
                start_i(w0 + 1, 1)

            @pl.when(g > 0)
            def _():
                wait_store(0)

            gather_store(w0, 0)

            @pl.when(w0 + 1 < nwin)
            def _():
                wait_i(1)

                @pl.when(w0 + 2 < nwin)
                def _():
                    start_i(w0 + 2, 0)

                @pl.when(g > 0)
                def _():
                    wait_store(1)

                gather_store(w0 + 1, 1)

            return carry

        lax.fori_loop(0, (nwin + 1) // 2, pair, 0)
        wait_store(0)
        if nwin > 1:
            wait_store(1)

    return k


@functools.lru_cache(maxsize=None)
def _sc_scatter_add_rows(V, Dw, B, W):
    b_per_w = B // _NW
    nwin = b_per_w // W
    assert b_per_w % W == 0 and W % 8 == 0 and W <= 128
    mesh = plsc.VectorSubcoreMesh(core_axis_name="c", subcore_axis_name="s")

    @functools.partial(
        pl.kernel,
        out_type=jax.ShapeDtypeStruct((_NC, V, Dw), jnp.float32),
        mesh=mesh,
        compiler_params=pltpu.CompilerParams(use_tc_tiling_on_sc=(Dw % 128 == 0)),
        scratch_types=[
            pltpu.VMEM((W,), jnp.int32),
            pltpu.VMEM((W,), jnp.int32),
            pltpu.VMEM((2, W, Dw), jnp.float32),
            pltpu.VMEM_SHARED((V, Dw), jnp.float32),
            pltpu.SemaphoreType.DMA,
            pltpu.SemaphoreType.DMA,
        ],
    )
    def k(upd_hbm, idx_hbm, zero_hbm, out_hbm, idx_v0, idx_v1, upd_v, acc_sh,
          semi, semu):
        cid = lax.axis_index("c")
        sid = lax.axis_index("s")
        wid = sid * _NC + cid
        idxv = (idx_v0, idx_v1)

        @pl.when(sid == 0)
        def _():
            pltpu.sync_copy(zero_hbm, acc_sh)

        plsc.subcore_barrier()

        def start_iu(w, b):
            base = wid * b_per_w + w * W
            pltpu.async_copy(idx_hbm.at[pl.ds(base, W)], idxv[b], semi)
            pltpu.async_copy(upd_hbm.at[pl.ds(base, W)], upd_v.at[b], semu)

        def wait_iu(b):
            pltpu.make_async_copy(idx_hbm.at[pl.ds(0, W)], idxv[b],
                                  semi).wait()
            pltpu.make_async_copy(upd_hbm.at[pl.ds(0, W)], upd_v.at[b],
                                  semu).wait()

        def scat(b):
            pltpu.sync_copy(upd_v.at[b], acc_sh.at[idxv[b]], add=True)

        start_iu(0, 0)

        def pair(g, carry):
            w0 = 2 * g
            wait_iu(0)

            @pl.when(w0 + 1 < nwin)
            def _():
                start_iu(w0 + 1, 1)

            scat(0)

            @pl.when(w0 + 1 < nwin)
            def _():
                wait_iu(1)

                @pl.when(w0 + 2 < nwin)
                def _():
                    start_iu(w0 + 2, 0)

                scat(1)

            return carry

        lax.fori_loop(0, (nwin + 1) // 2, pair, 0)
        plsc.subcore_barrier()

        @pl.when(sid == 0)
        def _():
            pltpu.sync_copy(acc_sh, out_hbm.at[cid])

    return k


_SR = 4096
_SC_ = 128
_S = _SR * _SC_
_NBITS = 19


def _sort_schedule():
    ds, sb = [], []
    for s in range(1, _NBITS + 1):
        d = 1 << (s - 1)
        while d >= 1:
            ds.append(d)
            sb.append(1 << s)
            d //= 2
    return np.array(ds, np.int32), np.array(sb, np.int32)


def _sort_body(score_ref, dsched_ref, out_ref, key_ref):
    rows = lax.broadcasted_iota(jnp.int32, (_SR, _SC_), 0)
    cols = lax.broadcasted_iota(jnp.int32, (_SR, _SC_), 1)
    ig = rows + _SR * cols
    b = pltpu.bitcast(score_ref[...], jnp.int32)
    key = jnp.where(b >= 0, jnp.int32(0x7FFFFFFF) - b, b) ^ jnp.int32(-2147483648)
    key_ref[...] = key
    out_ref[...] = ig

    nsteps = dsched_ref.shape[0] // 2

    def step(t, carry):
        d = dsched_ref[2 * t]
        sblk = dsched_ref[2 * t + 1]
        ai = key_ref[...]
        ix = out_ref[...]
        first = (ig & d) == 0
        asc = (ig & sblk) == 0
        keep_small = first == asc

        def row_case(ai, ix):
            return (
                pltpu.roll(ai, _SR - d, 0), pltpu.roll(ai, d, 0),
                pltpu.roll(ix, _SR - d, 0), pltpu.roll(ix, d, 0),
            )

        def col_case(ai, ix):
            m = d >> 12
            return (
                pltpu.roll(ai, _SC_ - m, 1), pltpu.roll(ai, m, 1),
                pltpu.roll(ix, _SC_ - m, 1), pltpu.roll(ix, m, 1),
            )

        fa, ba, fi, bi = lax.cond(d < _SR, row_case, col_case, ai, ix)
        pa = jnp.where(first, fa, ba)
        pi = jnp.where(first, fi, bi)
        mine_less = (ai < pa) | ((ai == pa) & (ix < pi))
        take = keep_small ^ mine_less
        key_ref[...] = jnp.where(take, pa, ai)
        out_ref[...] = jnp.where(take, pi, ix)
        return carry

    lax.fori_loop(0, nsteps, step, 0)


def _bitonic_argsort(score):
    pad = jnp.full((_S - _E,), -jnp.inf, jnp.float32)
    s2 = jnp.concatenate([score, pad]).reshape(_SC_, _SR).T
    ds, sb = _sort_schedule()
    sched = jnp.asarray(np.stack([ds, sb], 1).reshape(-1))
    idx2d, _ = pl.pallas_call(
        _sort_body,
        out_shape=(
            jax.ShapeDtypeStruct((_SR, _SC_), jnp.int32),
            jax.ShapeDtypeStruct((_SR, _SC_), jnp.int32),
        ),
        in_specs=[
            pl.BlockSpec(memory_space=pltpu.VMEM),
            pl.BlockSpec(memory_space=pltpu.SMEM),
        ],
        out_specs=(
            pl.BlockSpec(memory_space=pltpu.VMEM),
            pl.BlockSpec(memory_space=pltpu.VMEM),
        ),
    )(s2, sched)
    return idx2d.T.reshape(-1)


_BE2 = 8000


def _edge2_body(hsrc_ref, p_ref, g0_ref, g1_ref, rep_ref, out_ref):
    denom = g0_ref[...] + g1_ref[...] + jnp.float32(1e-16)
    alpha16 = p_ref[...] / denom
    afull = jnp.dot(alpha16, rep_ref[...], preferred_element_type=jnp.float32)
    out_ref[...] = hsrc_ref[...] * afull


def _edge2(hsrc, p16, gs0, gs1, rep):
    grid = _E // _BE2
    return pl.pallas_call(
        _edge2_body,
        grid=(grid,),
        in_specs=[
            pl.BlockSpec((_BE2, _D), lambda i: (i, 0)),
            pl.BlockSpec((_BE2, 16), lambda i: (i, 0)),
            pl.BlockSpec((_BE2, 16), lambda i: (i, 0)),
            pl.BlockSpec((_BE2, 16), lambda i: (i, 0)),
            pl.BlockSpec((16, _D), lambda i: (0, 0)),
        ],
        out_specs=pl.BlockSpec((_BE2, _D), lambda i: (i, 0)),
        out_shape=jax.ShapeDtypeStruct((_E, _D), jnp.float32),
    )(hsrc, p16, gs0, gs1, rep)


_BNF = 2000


def _final_body(ai_ref, bi_ref, aj_ref, bj_ref, wg_ref, bg_ref, w1_ref, w2_ref,
                out_ref):
    xi = ai_ref[0] + ai_ref[1] + bi_ref[...]
    xj = aj_ref[0] + aj_ref[1] + bj_ref[...]
    cat = jnp.concatenate([xi, xj], axis=1)
    g = jax.nn.sigmoid(
        jnp.dot(cat, wg_ref[...], preferred_element_type=jnp.float32)
        + bg_ref[...])
    fusion = (g * jnp.dot(xi, w1_ref[...], preferred_element_type=jnp.float32)
              + (1.0 - g) * jnp.dot(xj, w2_ref[...],
                                    preferred_element_type=jnp.float32))
    out_ref[0] = fusion + xi
    out_ref[1] = fusion + xj


def _final(acc_i, bout_i, acc_j, bout_j, Wg, bg, W1, W2):
    grid = _N // _BNF
    return pl.pallas_call(
        _final_body,
        grid=(grid,),
        in_specs=[
            pl.BlockSpec((2, _BNF, _D), lambda i: (0, i, 0)),
            pl.BlockSpec((1, _D), lambda i: (0, 0)),
            pl.BlockSpec((2, _BNF, _D), lambda i: (0, i, 0)),
            pl.BlockSpec((1, _D), lambda i: (0, 0)),
            pl.BlockSpec((2 * _D, _D), lambda i: (0, 0)),
            pl.BlockSpec((1, _D), lambda i: (0, 0)),
            pl.BlockSpec((_D, _D), lambda i: (0, 0)),
            pl.BlockSpec((_D, _D), lambda i: (0, 0)),
        ],
        out_specs=pl.BlockSpec((2, _BNF, _D), lambda i: (0, i, 0)),
        out_shape=jax.ShapeDtypeStruct((2, _N, _D), jnp.float32),
    )(acc_i, bout_i.reshape(1, _D), acc_j, bout_j.reshape(1, _D),
      Wg, bg.reshape(1, _D), W1, W2)


def _tree_sum(t):
    while t.shape[-1] > 1:
        t = t[..., 0::2] + t[..., 1::2]
    return t[..., 0]


def _fold_sum(t):
    while t.shape[-1] > 1:
        m = t.shape[-1] // 2
        t = t[..., :m] + t[..., m:]
    return t[..., 0]


def _node_embed(x, gamma, beta, Wx, bx):
    mu = jnp.mean(x, axis=0)
    var = jnp.var(x, axis=0)
    xn = (x - mu) / jnp.sqrt(var + 1e-5) * gamma + beta
    return xn @ Wx + bx


def _score_path(hsrc, hdst, asrc, adst, ea, We, ae):
    t1 = _tree_sum(hsrc.reshape(_E, _H, _DH) * asrc)
    t2 = _tree_sum(hdst.reshape(_E, _H, _DH) * adst)
    he = (ea @ We).reshape(_E, _H, _DH)
    t3 = _fold_sum(he * ae)
    logits = jax.nn.leaky_relu((t1 + t2) + t3, 0.2)
    score = _fold_sum(logits) / jnp.float32(8.0)
    p8 = jnp.exp(logits)
    p16 = jnp.concatenate([p8, p8], axis=1)
    return p16, score


_W = 80


def kernel(x_intra, edge_index_intra, edge_attr_intra, batch_ei_intra,
           x_inter, edge_index_inter, edge_attr_inter, batch_ei_inter,
           gamma_i, beta_i, Wx_i, bx_i, We_i, asrc_i, adst_i, ae_i, bout_i,
           gamma_j, beta_j, Wx_j, bx_j, We_j, asrc_j, adst_j, ae_j, bout_j,
           Wg, bg, W1, W2):
    rep16 = np.zeros((16, _D), np.float32)
    for hh in range(_H):
        rep16[hh, hh * _DH:(hh + 1) * _DH] = 1.0
    rep16 = jnp.asarray(rep16)

    src_i, dst_i = edge_index_intra[0], edge_index_intra[1]
    src_j, dst_j = edge_index_inter[0], edge_index_inter[1]

    h_i = _node_embed(x_intra, gamma_i, beta_i, Wx_i, bx_i)
    h_j = _node_embed(x_inter, gamma_j, beta_j, Wx_j, bx_j)

    hsrc_i = _sc_gather_rows(_N, _D, _E, "float32", _W)(h_i, src_i)
    hdst_i = _sc_gather_rows(_N, _D, _E, "float32", _W)(h_i, dst_i)
    hsrc_j = _sc_gather_rows(_N, _D, _E, "float32", _W)(h_j, src_j)
    hdst_j = _sc_gather_rows(_N, _D, _E, "float32", _W)(h_j, dst_j)

    p16_i, score_i = _score_path(hsrc_i, hdst_i, asrc_i, adst_i,
                                 edge_attr_intra, We_i, ae_i)
    p16_j, score_j = _score_path(hsrc_j, hdst_j, asrc_j, adst_j,
                                 edge_attr_inter, We_j, ae_j)

    zeros16 = jnp.zeros((_N, 16), jnp.float32)
    ssum_i = _sc_scatter_add_rows(_N, 16, _E, _W)(p16_i, dst_i, zeros16)
    ssum_j = _sc_scatter_add_rows(_N, 16, _E, _W)(p16_j, dst_j, zeros16)
    gs0_i = _sc_gather_rows(_N, 16, _E, "float32", _W)(ssum_i[0], dst_i)
    gs1_i = _sc_gather_rows(_N, 16, _E, "float32", _W)(ssum_i[1], dst_i)
    gs0_j = _sc_gather_rows(_N, 16, _E, "float32", _W)(ssum_j[0], dst_j)
    gs1_j = _sc_gather_rows(_N, 16, _E, "float32", _W)(ssum_j[1], dst_j)

    zeros128 = jnp.zeros((_N, _D), jnp.float32)
    upd_i = _edge2(hsrc_i, p16_i, gs0_i, gs1_i, rep16)
    acc_i = _sc_scatter_add_rows(_N, _D, _E, _W)(
        upd_i, dst_i, zeros128)
    upd_j = _edge2(hsrc_j, p16_j, gs0_j, gs1_j, rep16)
    acc_j = _sc_scatter_add_rows(_N, _D, _E, _W)(
        upd_j, dst_j, zeros128)

    out = _final(acc_i, bout_i, acc_j, bout_j, Wg, bg, W1, W2)

    idx_i = _bitonic_argsort(score_i)[:_K]
    idx_j = _bitonic_argsort(score_j)[:_K]
    pad14_i = jnp.concatenate([edge_index_intra.T.astype(jnp.int32),
                               jnp.zeros((_E, 14), jnp.int32)], axis=1)
    pad14_j = jnp.concatenate([edge_index_inter.T.astype(jnp.int32),
                               jnp.zeros((_E, 14), jnp.int32)], axis=1)
    ei_i = _sc_gather_rows(_E, 16, _K, "int32", _W)(pad14_i, idx_i)[:, :2].T
    ei_j = _sc_gather_rows(_E, 16, _K, "int32", _W)(pad14_j, idx_j)[:, :2].T

    return (out, ei_i, ei_j)


# speedup vs baseline: 1.0548x; 1.0002x over previous
import functools

import jax
import jax.numpy as jnp
import numpy as np
from jax import lax
from jax.experimental import pallas as pl
from jax.experimental.pallas import tpu as pltpu
from jax.experimental.pallas import tpu_sc as plsc

_N = 10000
_E = 320000
_D = 128
_DE = 16
_H = 8
_DH = _D // _H
_K = int(np.ceil(0.8 * _E))

_NC = 2
_NS = 16
_NW = _NC * _NS


@functools.lru_cache(maxsize=None)
def _sc_gather_rows(V, Dw, B, dtype_name, W):
    dtype = jnp.dtype(dtype_name)
    b_per_w = B // _NW
    nwin = b_per_w // W
    assert b_per_w % W == 0 and W % 8 == 0 and W <= 128
    mesh = plsc.VectorSubcoreMesh(core_axis_name="c", subcore_axis_name="s")

    @functools.partial(
        pl.kernel,
        out_type=jax.ShapeDtypeStruct((B, Dw), dtype),
        mesh=mesh,
        compiler_params=pltpu.CompilerParams(use_tc_tiling_on_sc=(Dw % 128 == 0)),
        scratch_types=[
            pltpu.VMEM((W,), jnp.int32),
            pltpu.VMEM((W,), jnp.int32),
            pltpu.VMEM((2, W, Dw), dtype),
            pltpu.SemaphoreType.DMA,
            pltpu.SemaphoreType.DMA,
            pltpu.SemaphoreType.DMA,
            pltpu.SemaphoreType.DMA,
        ],
    )
    def k(table_hbm, idx_hbm, out_hbm, idx_v0, idx_v1, rows_v,
          semi, semg, semo0, semo1):
        wid = lax.axis_index("s") * _NC + lax.axis_index("c")
        idxv = (idx_v0, idx_v1)
        semo = (semo0, semo1)

        def start_i(w, b):
            base = wid * b_per_w + w * W
            pltpu.async_copy(idx_hbm.at[pl.ds(base, W)], idxv[b], semi)

        def wait_i(b):
            pltpu.make_async_copy(idx_hbm.at[pl.ds(0, W)], idxv[b],
                                  semi).wait()

        def gather_store(w, b):
            base = wid * b_per_w + w * W
            pltpu.async_copy(table_hbm.at[idxv[b]], rows_v.at[b], semg).wait()
            pltpu.async_copy(rows_v.at[b], out_hbm.at[pl.ds(base, W)], semo[b])

        def wait_store(b):
            pltpu.make_async_copy(rows_v.at[b], out_hbm.at[pl.ds(0, W)],
                                  semo[b]).wait()

        start_i(0, 0)

        def pair(g, carry):
            w0 = 2 * g
            wait_i(0)

            @pl.when(w0 + 1 < nwin)
            def _():
                start_i(w0 + 1, 1)

            @pl.when(g > 0)
            def _():
                wait_store(0)

            gather_store(w0, 0)

            @pl.when(w0 + 1 < nwin)
            def _():
                wait_i(1)

                @pl.when(w0 + 2 < nwin)
                def _():
                    start_i(w0 + 2, 0)

                @pl.when(g > 0)
                def _():
                    wait_store(1)

                gather_store(w0 + 1, 1)

            return carry

        lax.fori_loop(0, (nwin + 1) // 2, pair, 0)
        wait_store(0)
        if nwin > 1:
            wait_store(1)

    return k


@functools.lru_cache(maxsize=None)
def _sc_scatter_add_rows(V, Dw, B, W):
    b_per_w = B // _NW
    nwin = b_per_w // W
    assert b_per_w % W == 0 and W % 8 == 0 and W <= 128
    mesh = plsc.VectorSubcoreMesh(core_axis_name="c", subcore_axis_name="s")

    @functools.partial(
        pl.kernel,
        out_type=jax.ShapeDtypeStruct((_NC, V, Dw), jnp.float32),
        mesh=mesh,
        compiler_params=pltpu.CompilerParams(use_tc_tiling_on_sc=(Dw % 128 == 0)),
        scratch_types=[
            pltpu.VMEM((W,), jnp.int32),
            pltpu.VMEM((W,), jnp.int32),
            pltpu.VMEM((2, W, Dw), jnp.float32),
            pltpu.VMEM_SHARED((V, Dw), jnp.float32),
            pltpu.SemaphoreType.DMA,
            pltpu.SemaphoreType.DMA,
        ],
    )
    def k(upd_hbm, idx_hbm, zero_hbm, out_hbm, idx_v0, idx_v1, upd_v, acc_sh,
          semi, semu):
        cid = lax.axis_index("c")
        sid = lax.axis_index("s")
        wid = sid * _NC + cid
        idxv = (idx_v0, idx_v1)

        @pl.when(sid == 0)
        def _():
            pltpu.sync_copy(zero_hbm, acc_sh)

        plsc.subcore_barrier()

        def start_iu(w, b):
            base = wid * b_per_w + w * W
            pltpu.async_copy(idx_hbm.at[pl.ds(base, W)], idxv[b], semi)
            pltpu.async_copy(upd_hbm.at[pl.ds(base, W)], upd_v.at[b], semu)

        def wait_iu(b):
            pltpu.make_async_copy(idx_hbm.at[pl.ds(0, W)], idxv[b],
                                  semi).wait()
            pltpu.make_async_copy(upd_hbm.at[pl.ds(0, W)], upd_v.at[b],
                                  semu).wait()

        def scat(b):
            pltpu.sync_copy(upd_v.at[b], acc_sh.at[idxv[b]], add=True)

        start_iu(0, 0)

        def pair(g, carry):
            w0 = 2 * g
            wait_iu(0)

            @pl.when(w0 + 1 < nwin)
            def _():
                start_iu(w0 + 1, 1)

            scat(0)

            @pl.when(w0 + 1 < nwin)
            def _():
                wait_iu(1)

                @pl.when(w0 + 2 < nwin)
                def _():
                    start_iu(w0 + 2, 0)

                scat(1)

            return carry

        lax.fori_loop(0, (nwin + 1) // 2, pair, 0)
        plsc.subcore_barrier()

        @pl.when(sid == 0)
        def _():
            pltpu.sync_copy(acc_sh, out_hbm.at[cid])

    return k


_SR = 4096
_SC_ = 128
_S = _SR * _SC_
_NBITS = 19


def _sort_schedule():
    ds, sb = [], []
    for s in range(1, _NBITS + 1):
        d = 1 << (s - 1)
        while d >= 1:
            ds.append(d)
            sb.append(1 << s)
            d //= 2
    return np.array(ds, np.int32), np.array(sb, np.int32)


def _sort_body(score_ref, dsched_ref, out_ref, key_ref):
    rows = lax.broadcasted_iota(jnp.int32, (_SR, _SC_), 0)
    cols = lax.broadcasted_iota(jnp.int32, (_SR, _SC_), 1)
    ig = rows + _SR * cols
    b = pltpu.bitcast(score_ref[...], jnp.int32)
    key = jnp.where(b >= 0, jnp.int32(0x7FFFFFFF) - b, b) ^ jnp.int32(-2147483648)
    key_ref[...] = key
    out_ref[...] = ig

    nsteps = dsched_ref.shape[0] // 2

    def step(t, carry):
        d = dsched_ref[2 * t]
        sblk = dsched_ref[2 * t + 1]
        ai = key_ref[...]
        ix = out_ref[...]
        first = (ig & d) == 0
        asc = (ig & sblk) == 0
        keep_small = first == asc

        def row_case(ai, ix):
            return (
                pltpu.roll(ai, _SR - d, 0), pltpu.roll(ai, d, 0),
                pltpu.roll(ix, _SR - d, 0), pltpu.roll(ix, d, 0),
            )

        def col_case(ai, ix):
            m = d >> 12
            return (
                pltpu.roll(ai, _SC_ - m, 1), pltpu.roll(ai, m, 1),
                pltpu.roll(ix, _SC_ - m, 1), pltpu.roll(ix, m, 1),
            )

        fa, ba, fi, bi = lax.cond(d < _SR, row_case, col_case, ai, ix)
        pa = jnp.where(first, fa, ba)
        pi = jnp.where(first, fi, bi)
        mine_less = (ai < pa) | ((ai == pa) & (ix < pi))
        take = keep_small ^ mine_less
        key_ref[...] = jnp.where(take, pa, ai)
        out_ref[...] = jnp.where(take, pi, ix)
        return carry

    lax.fori_loop(0, nsteps, step, 0)


def _bitonic_argsort(score):
    pad = jnp.full((_S - _E,), -jnp.inf, jnp.float32)
    s2 = jnp.concatenate([score, pad]).reshape(_SC_, _SR).T
    ds, sb = _sort_schedule()
    sched = jnp.asarray(np.stack([ds, sb], 1).reshape(-1))
    idx2d, _ = pl.pallas_call(
        _sort_body,
        out_shape=(
            jax.ShapeDtypeStruct((_SR, _SC_), jnp.int32),
            jax.ShapeDtypeStruct((_SR, _SC_), jnp.int32),
        ),
        in_specs=[
            pl.BlockSpec(memory_space=pltpu.VMEM),
            pl.BlockSpec(memory_space=pltpu.SMEM),
        ],
        out_specs=(
            pl.BlockSpec(memory_space=pltpu.VMEM),
            pl.BlockSpec(memory_space=pltpu.VMEM),
        ),
    )(s2, sched)
    return idx2d.T.reshape(-1)


_BE2 = 8000


def _edge2_body(hsrc_ref, p_ref, g0_ref, g1_ref, rep_ref, out_ref):
    denom = g0_ref[...] + g1_ref[...] + jnp.float32(1e-16)
    alpha16 = p_ref[...] / denom
    afull = jnp.dot(alpha16, rep_ref[...], preferred_element_type=jnp.float32)
    out_ref[...] = hsrc_ref[...] * afull


def _edge2(hsrc, p16, gs0, gs1, rep):
    grid = _E // _BE2
    return pl.pallas_call(
        _edge2_body,
        grid=(grid,),
        in_specs=[
            pl.BlockSpec((_BE2, _D), lambda i: (i, 0)),
            pl.BlockSpec((_BE2, 16), lambda i: (i, 0)),
            pl.BlockSpec((_BE2, 16), lambda i: (i, 0)),
            pl.BlockSpec((_BE2, 16), lambda i: (i, 0)),
            pl.BlockSpec((16, _D), lambda i: (0, 0)),
        ],
        out_specs=pl.BlockSpec((_BE2, _D), lambda i: (i, 0)),
        out_shape=jax.ShapeDtypeStruct((_E, _D), jnp.float32),
    )(hsrc, p16, gs0, gs1, rep)


_BNF = 2000


def _final_body(ai_ref, bi_ref, aj_ref, bj_ref, wg_ref, bg_ref, w1_ref, w2_ref,
                out_ref):
    xi = ai_ref[0] + ai_ref[1] + bi_ref[...]
    xj = aj_ref[0] + aj_ref[1] + bj_ref[...]
    cat = jnp.concatenate([xi, xj], axis=1)
    g = jax.nn.sigmoid(
        jnp.dot(cat, wg_ref[...], preferred_element_type=jnp.float32)
        + bg_ref[...])
    fusion = (g * jnp.dot(xi, w1_ref[...], preferred_element_type=jnp.float32)
              + (1.0 - g) * jnp.dot(xj, w2_ref[...],
                                    preferred_element_type=jnp.float32))
    out_ref[0] = fusion + xi
    out_ref[1] = fusion + xj


def _final(acc_i, bout_i, acc_j, bout_j, Wg, bg, W1, W2):
    grid = _N // _BNF
    return pl.pallas_call(
        _final_body,
        grid=(grid,),
        in_specs=[
            pl.BlockSpec((2, _BNF, _D), lambda i: (0, i, 0)),
            pl.BlockSpec((1, _D), lambda i: (0, 0)),
            pl.BlockSpec((2, _BNF, _D), lambda i: (0, i, 0)),
            pl.BlockSpec((1, _D), lambda i: (0, 0)),
            pl.BlockSpec((2 * _D, _D), lambda i: (0, 0)),
            pl.BlockSpec((1, _D), lambda i: (0, 0)),
            pl.BlockSpec((_D, _D), lambda i: (0, 0)),
            pl.BlockSpec((_D, _D), lambda i: (0, 0)),
        ],
        out_specs=pl.BlockSpec((2, _BNF, _D), lambda i: (0, i, 0)),
        out_shape=jax.ShapeDtypeStruct((2, _N, _D), jnp.float32),
    )(acc_i, bout_i.reshape(1, _D), acc_j, bout_j.reshape(1, _D),
      Wg, bg.reshape(1, _D), W1, W2)


def _tree_sum(t):
    while t.shape[-1] > 1:
        t = t[..., 0::2] + t[..., 1::2]
    return t[..., 0]


def _fold_sum(t):
    while t.shape[-1] > 1:
        m = t.shape[-1] // 2
        t = t[..., :m] + t[..., m:]
    return t[..., 0]


def _node_embed(x, gamma, beta, Wx, bx):
    mu = jnp.mean(x, axis=0)
    var = jnp.var(x, axis=0)
    xn = (x - mu) / jnp.sqrt(var + 1e-5) * gamma + beta
    return xn @ Wx + bx


def _score_path(hsrc, hdst, asrc, adst, ea, We, ae):
    t1 = _tree_sum(hsrc.reshape(_E, _H, _DH) * asrc)
    t2 = _tree_sum(hdst.reshape(_E, _H, _DH) * adst)
    he = (ea @ We).reshape(_E, _H, _DH)
    t3 = _fold_sum(he * ae)
    logits = jax.nn.leaky_relu((t1 + t2) + t3, 0.2)
    score = _fold_sum(logits) / jnp.float32(8.0)
    p8 = jnp.exp(logits)
    p16 = jnp.concatenate([p8, p8], axis=1)
    return p16, score


_W = 80


def kernel(x_intra, edge_index_intra, edge_attr_intra, batch_ei_intra,
           x_inter, edge_index_inter, edge_attr_inter, batch_ei_inter,
           gamma_i, beta_i, Wx_i, bx_i, We_i, asrc_i, adst_i, ae_i, bout_i,
           gamma_j, beta_j, Wx_j, bx_j, We_j, asrc_j, adst_j, ae_j, bout_j,
           Wg, bg, W1, W2):
    rep16 = np.zeros((16, _D), np.float32)
    for hh in range(_H):
        rep16[hh, hh * _DH:(hh + 1) * _DH] = 1.0
    rep16 = jnp.asarray(rep16)

    src_i, dst_i = edge_index_intra[0], edge_index_intra[1]
    src_j, dst_j = edge_index_inter[0], edge_index_inter[1]

    h_i = _node_embed(x_intra, gamma_i, beta_i, Wx_i, bx_i)
    h_j = _node_embed(x_inter, gamma_j, beta_j, Wx_j, bx_j)

    hsrc_i = _sc_gather_rows(_N, _D, _E, "float32", _W)(h_i, src_i)
    hdst_i = _sc_gather_rows(_N, _D, _E, "float32", _W)(h_i, dst_i)
    hsrc_j = _sc_gather_rows(_N, _D, _E, "float32", _W)(h_j, src_j)
    hdst_j = _sc_gather_rows(_N, _D, _E, "float32", _W)(h_j, dst_j)

    p16_i, score_i = _score_path(hsrc_i, hdst_i, asrc_i, adst_i,
                                 edge_attr_intra, We_i, ae_i)
    p16_j, score_j = _score_path(hsrc_j, hdst_j, asrc_j, adst_j,
                                 edge_attr_inter, We_j, ae_j)

    zeros16 = jnp.zeros((_N, 16), jnp.float32)
    ssum_i = _sc_scatter_add_rows(_N, 16, _E, _W)(p16_i, dst_i, zeros16)
    ssum_j = _sc_scatter_add_rows(_N, 16, _E, _W)(p16_j, dst_j, zeros16)
    gs0_i = _sc_gather_rows(_N, 16, _E, "float32", _W)(ssum_i[0], dst_i)
    gs1_i = _sc_gather_rows(_N, 16, _E, "float32", _W)(ssum_i[1], dst_i)
    gs0_j = _sc_gather_rows(_N, 16, _E, "float32", _W)(ssum_j[0], dst_j)
    gs1_j = _sc_gather_rows(_N, 16, _E, "float32", _W)(ssum_j[1], dst_j)

    zeros128 = jnp.zeros((_N, _D), jnp.float32)
    upd_i = _edge2(hsrc_i, p16_i, gs0_i, gs1_i, rep16)
    acc_i = _sc_scatter_add_rows(_N, _D, _E, _W)(
        upd_i, dst_i, zeros128)
    upd_j = _edge2(hsrc_j, p16_j, gs0_j, gs1_j, rep16)
    acc_j = _sc_scatter_add_rows(_N, _D, _E, _W)(
        upd_j, dst_j, zeros128)

    out = _final(acc_i, bout_i, acc_j, bout_j, Wg, bg, W1, W2)

    idx_i = _bitonic_argsort(score_i)[:_K]
    idx_j = _bitonic_argsort(score_j)[:_K]
    pad14_i = jnp.concatenate([edge_index_intra.T.astype(jnp.int32),
                               jnp.zeros((_E, 14), jnp.int32)], axis=1)
    pad14_j = jnp.concatenate([edge_index_inter.T.astype(jnp.int32),
                               jnp.zeros((_E, 14), jnp.int32)], axis=1)
    ei_i = _sc_gather_rows(_E, 16, _K, "int32", _W)(pad14_i, idx_i)[:, :2].T
    ei_j = _sc_gather_rows(_E, 16, _K, "int32", _W)(pad14_j, idx_j)[:, :2].T

    return (out, ei_i, ei_j)
